# baseline (speedup 1.0000x reference)
import jax
import jax.numpy as jnp
from jax.experimental import pallas as pl
from jax.experimental.pallas import tpu as pltpu

H = 512
W = 512
WS = 8
TOP_K = 500
THR = 0.01
HP = jax.lax.Precision.DEFAULT


def _norm_body(img_ref, out_ref):
    x = img_ref[...]
    mu = jnp.mean(x)
    var = jnp.mean((x - mu) * (x - mu))
    out_ref[...] = (x - mu) / jnp.sqrt(var + 1e-5)


def _mm_relu_body(a_ref, w_ref, b_ref, o_ref):
    y = jnp.dot(a_ref[...], w_ref[...], precision=HP,
                preferred_element_type=jnp.float32) + b_ref[...]
    o_ref[...] = jnp.maximum(y, 0.0)


def _mm_relu(a, w, b, bm):
    M, K = a.shape
    N = w.shape[1]
    return pl.pallas_call(
        _mm_relu_body,
        grid=(M // bm,),
        in_specs=[pl.BlockSpec((bm, K), lambda i: (i, 0)),
                  pl.BlockSpec((K, N), lambda i: (0, 0)),
                  pl.BlockSpec((1, N), lambda i: (0, 0))],
        out_specs=pl.BlockSpec((bm, N), lambda i: (i, 0)),
        out_shape=_f32((M, N)),
    )(a, w, b)


def _heads_body(x3_ref, u_ref, wf_ref, bf_ref, wk1_ref, bk1_ref,
                wk2_ref, bk2_ref, wk3_ref, bk3_ref, wh_ref, bh_ref,
                sc_ref, feat_ref):
    x3 = x3_ref[...]
    u = u_ref[...]
    M = jnp.dot(x3, wf_ref[...], precision=HP,
                preferred_element_type=jnp.float32) + bf_ref[...]
    Mn = M / (jnp.sqrt(jnp.sum(M * M, axis=1, keepdims=True)) + 1e-8)
    h = jnp.dot(x3, wh_ref[...], precision=HP,
                preferred_element_type=jnp.float32) + bh_ref[...]
    hrel = 1.0 / (1.0 + jnp.exp(-h))
    k = jnp.dot(u, wk1_ref[...], precision=HP,
                preferred_element_type=jnp.float32) + bk1_ref[...]
    k = jnp.maximum(k, 0.0)
    k = jnp.dot(k, wk2_ref[...], precision=HP,
                preferred_element_type=jnp.float32) + bk2_ref[...]
    k = jnp.maximum(k, 0.0)
    k = jnp.dot(k, wk3_ref[...], precision=HP,
                preferred_element_type=jnp.float32) + bk3_ref[...]
    k = k - jnp.max(k, axis=1, keepdims=True)
    e = jnp.exp(k)
    sc = e / jnp.sum(e, axis=1, keepdims=True)
    sc_ref[...] = sc[:, :64]
    feat_ref[...] = jnp.concatenate([Mn, hrel], axis=1)


def _nms_topk_body(hp_ref, vals_ref, idxs_ref, smap_ref):
    hp = hp_ref[...]
    cm = hp[:, 0:512]
    for dx in range(1, 5):
        cm = jnp.maximum(cm, hp[:, dx:dx + 512])
    pooled = cm[0:512, :]
    for dy in range(1, 5):
        pooled = jnp.maximum(pooled, cm[dy:dy + 512, :])
    heat = hp[2:514, 2:514]
    keep = jnp.logical_and(heat == pooled, heat > THR)
    smap = jnp.where(keep, heat, 0.0)
    smap_ref[...] = smap
    rowmax = jnp.max(smap, axis=1).reshape(1, 512)

    li = jax.lax.broadcasted_iota(jnp.int32, (1, 512), 1)

    def step(t, carry):
        rowmax, vals, idxs = carry
        m = jnp.max(rowmax)
        b = jnp.min(jnp.where(rowmax == m, li, 512))
        row = smap_ref[pl.ds(b, 1), :]
        p = jnp.min(jnp.where(row == m, li, 512))
        idx = b * 512 + p
        vals = jnp.where(li == t, m, vals)
        idxs = jnp.where(li == t, idx, idxs)
        newrow = jnp.where(li == p, -1.0, row)
        smap_ref[pl.ds(b, 1), :] = newrow
        rowmax = jnp.where(li == b, jnp.max(newrow), rowmax)
        return rowmax, vals, idxs

    vals0 = jnp.zeros((1, 512), jnp.float32)
    idxs0 = jnp.zeros((1, 512), jnp.int32)
    rowmax, vals, idxs = jax.lax.fori_loop(
        0, TOP_K, step, (rowmax, vals0, idxs0))
    vals_ref[...] = vals
    idxs_ref[...] = idxs


_SAMPLE_BLOCKS = 4
_SAMPLE_BK = 4096 // _SAMPLE_BLOCKS


def _sample_body(idx_ref, val_ref, feat_ref, kpts_ref, desc_ref, sc_ref,
                 acc_ref):
    j = pl.program_id(0)
    idx = idx_ref[...]
    vals = val_ref[...]
    kx = idx % W
    ky = idx // W
    x0 = kx // WS
    y0 = ky // WS
    wx = (kx % WS).astype(jnp.float32) / WS
    wy = (ky % WS).astype(jnp.float32) / WS
    x1 = jnp.minimum(x0 + 1, 63)
    y1 = jnp.minimum(y0 + 1, 63)
    c00 = y0 * 64 + x0
    c01 = y0 * 64 + x1
    c10 = y1 * 64 + x0
    c11 = y1 * 64 + x1
    w00 = (1.0 - wx) * (1.0 - wy)
    w01 = wx * (1.0 - wy)
    w10 = (1.0 - wx) * wy
    w11 = wx * wy
    ci = jax.lax.broadcasted_iota(jnp.int32, (512, _SAMPLE_BK), 1) \
        + j * _SAMPLE_BK
    Wm = jnp.where(ci == c00, w00, 0.0)
    Wm = Wm + jnp.where(ci == c01, w01, 0.0)
    Wm = Wm + jnp.where(ci == c10, w10, 0.0)
    Wm = Wm + jnp.where(ci == c11, w11, 0.0)
    part = jnp.dot(Wm, feat_ref[...], precision=HP,
                   preferred_element_type=jnp.float32)

    @pl.when(j == 0)
    def _():
        acc_ref[...] = part
        kpts_ref[...] = jnp.concatenate(
            [kx.astype(jnp.float32), ky.astype(jnp.float32)], axis=1)

    @pl.when(j > 0)
    def _():
        acc_ref[...] = acc_ref[...] + part

    @pl.when(j == _SAMPLE_BLOCKS - 1)
    def _():
        S = acc_ref[...]
        d = S[:, :64]
        rel = S[:, 64:65]
        dn = d / (jnp.sqrt(jnp.sum(d * d, axis=1, keepdims=True)) + 1e-8)
        sc = vals * rel
        sc = jnp.where(idx == 0, -1.0, sc)
        desc_ref[...] = dn
        sc_ref[...] = sc


def _f32(shape):
    return jax.ShapeDtypeStruct(shape, jnp.float32)


def kernel(grayscale_image, w1, b1, w2, b2, w3, b3, wf, bf,
           wk1, bk1, wk2, bk2, wk3, bk3, wh, bh):
    img = grayscale_image.reshape(H, W)

    xn = pl.pallas_call(_norm_body, out_shape=_f32((H, W)))(img)

    xp = jnp.pad(xn, ((0, 1), (0, 1)))
    taps1 = [xp[dy:dy + 511:2, dx:dx + 511:2]
             for dy in range(3) for dx in range(3)]
    p1 = jnp.stack(taps1, axis=-1).reshape(256 * 256, 9)
    W1 = w1.reshape(24, 9).T
    y1 = _mm_relu(p1, W1, b1.reshape(1, 24), 4096)

    x1 = y1.reshape(256, 256, 24)
    x1p = jnp.pad(x1, ((0, 1), (0, 1), (0, 0)))
    taps2 = [x1p[dy:dy + 255:2, dx:dx + 255:2, :]
             for dy in range(3) for dx in range(3)]
    p2 = jnp.stack(taps2, axis=2).reshape(128 * 128, 9 * 24)
    W2 = w2.transpose(2, 3, 1, 0).reshape(9 * 24, 24)
    y2 = _mm_relu(p2, W2, b2.reshape(1, 24), 4096)

    x2 = y2.reshape(128, 128, 24)
    x2p = jnp.pad(x2, ((0, 1), (0, 1), (0, 0)))
    taps3 = [x2p[dy:dy + 127:2, dx:dx + 127:2, :]
             for dy in range(3) for dx in range(3)]
    p3 = jnp.stack(taps3, axis=2).reshape(64 * 64, 9 * 24)
    W3 = w3.transpose(2, 3, 1, 0).reshape(9 * 24, 64)
    x3 = _mm_relu(p3, W3, b3.reshape(1, 64), 4096)

    u = xn.reshape(64, 8, 64, 8).transpose(1, 3, 0, 2).reshape(64, 4096).T

    Wf = wf.reshape(64, 64).T
    Wk1 = wk1.reshape(64, 64).T
    Wk2 = wk2.reshape(64, 64).T
    Wk3 = wk3.reshape(65, 64).T
    Wh = wh.reshape(1, 64).T
    sc64, feat = pl.pallas_call(
        _heads_body,
        out_shape=(_f32((4096, 64)), _f32((4096, 65))),
    )(x3, u, Wf, bf.reshape(1, 64), Wk1, bk1.reshape(1, 64),
      Wk2, bk2.reshape(1, 64), Wk3, bk3.reshape(1, 65), Wh, bh.reshape(1, 1))

    heat = sc64.reshape(64, 64, 8, 8).transpose(0, 2, 1, 3).reshape(H, W)
    hpad = jnp.pad(heat, 2, constant_values=-jnp.inf)

    vals, idxs = pl.pallas_call(
        _nms_topk_body,
        out_shape=(_f32((1, 512)), jax.ShapeDtypeStruct((1, 512), jnp.int32)),
        scratch_shapes=[pltpu.VMEM((512, 512), jnp.float32)],
    )(hpad)

    kpts, desc, scores = pl.pallas_call(
        _sample_body,
        grid=(_SAMPLE_BLOCKS,),
        in_specs=[pl.BlockSpec((512, 1), lambda j: (0, 0)),
                  pl.BlockSpec((512, 1), lambda j: (0, 0)),
                  pl.BlockSpec((_SAMPLE_BK, 65), lambda j: (j, 0))],
        out_specs=(pl.BlockSpec((512, 2), lambda j: (0, 0)),
                   pl.BlockSpec((512, 64), lambda j: (0, 0)),
                   pl.BlockSpec((512, 1), lambda j: (0, 0))),
        out_shape=(_f32((512, 2)), _f32((512, 64)), _f32((512, 1))),
        scratch_shapes=[pltpu.VMEM((512, 65), jnp.float32)],
    )(idxs.reshape(512, 1), vals.reshape(512, 1), feat)

    return kpts[:TOP_K], desc[:TOP_K], scores[:TOP_K, 0]


# speedup vs baseline: 1.0507x; 1.0507x over previous
import jax
import jax.numpy as jnp
from jax.experimental import pallas as pl
from jax.experimental.pallas import tpu as pltpu

H = 512
W = 512
WS = 8
TOP_K = 500
THR = 0.01
HP = jax.lax.Precision.DEFAULT


def _norm_body(img_ref, out_ref):
    x = img_ref[...]
    mu = jnp.mean(x)
    var = jnp.mean((x - mu) * (x - mu))
    out_ref[...] = (x - mu) / jnp.sqrt(var + 1e-5)


def _mm_relu_body(a_ref, w_ref, b_ref, o_ref):
    y = jnp.dot(a_ref[...], w_ref[...], precision=HP,
                preferred_element_type=jnp.float32) + b_ref[...]
    o_ref[...] = jnp.maximum(y, 0.0)


def _mm_relu(a, w, b, bm):
    M, K = a.shape
    N = w.shape[1]
    return pl.pallas_call(
        _mm_relu_body,
        grid=(M // bm,),
        in_specs=[pl.BlockSpec((bm, K), lambda i: (i, 0)),
                  pl.BlockSpec((K, N), lambda i: (0, 0)),
                  pl.BlockSpec((1, N), lambda i: (0, 0))],
        out_specs=pl.BlockSpec((bm, N), lambda i: (i, 0)),
        out_shape=_f32((M, N)),
    )(a, w, b)


def _heads_body(x3_ref, u_ref, wf_ref, bf_ref, wk1_ref, bk1_ref,
                wk2_ref, bk2_ref, wk3_ref, bk3_ref, wh_ref, bh_ref,
                sc_ref, feat_ref):
    x3 = x3_ref[...]
    u = u_ref[...]
    M = jnp.dot(x3, wf_ref[...], precision=HP,
                preferred_element_type=jnp.float32) + bf_ref[...]
    Mn = M / (jnp.sqrt(jnp.sum(M * M, axis=1, keepdims=True)) + 1e-8)
    h = jnp.dot(x3, wh_ref[...], precision=HP,
                preferred_element_type=jnp.float32) + bh_ref[...]
    hrel = 1.0 / (1.0 + jnp.exp(-h))
    k = jnp.dot(u, wk1_ref[...], precision=HP,
                preferred_element_type=jnp.float32) + bk1_ref[...]
    k = jnp.maximum(k, 0.0)
    k = jnp.dot(k, wk2_ref[...], precision=HP,
                preferred_element_type=jnp.float32) + bk2_ref[...]
    k = jnp.maximum(k, 0.0)
    k = jnp.dot(k, wk3_ref[...], precision=HP,
                preferred_element_type=jnp.float32) + bk3_ref[...]
    k = k - jnp.max(k, axis=1, keepdims=True)
    e = jnp.exp(k)
    sc = e / jnp.sum(e, axis=1, keepdims=True)
    sc_ref[...] = sc[:, :64]
    feat_ref[...] = jnp.concatenate([Mn, hrel], axis=1)


def _nms_topk_body(hp_ref, vals_ref, idxs_ref, smap_ref):
    hp = hp_ref[...]
    cm = hp[:, 0:512]
    for dx in range(1, 5):
        cm = jnp.maximum(cm, hp[:, dx:dx + 512])
    pooled = cm[0:512, :]
    for dy in range(1, 5):
        pooled = jnp.maximum(pooled, cm[dy:dy + 512, :])
    heat = hp[2:514, 2:514]
    keep = jnp.logical_and(heat == pooled, heat > THR)
    smap = jnp.where(keep, heat, 0.0)
    smap_ref[...] = smap
    rowmax = jnp.max(smap, axis=1).reshape(1, 512)

    li = jax.lax.broadcasted_iota(jnp.int32, (1, 512), 1)

    def step(t, carry):
        rowmax, vals, idxs = carry
        m = jnp.max(rowmax, axis=1, keepdims=True)
        bv = jnp.min(jnp.where(rowmax == m, li, 512), axis=1,
                     keepdims=True)
        b = jnp.sum(bv)
        row = smap_ref[pl.ds(b, 1), :]
        pv = jnp.min(jnp.where(row == m, li, 512), axis=1,
                     keepdims=True)
        vals = jnp.where(li == t, m, vals)
        idxs = jnp.where(li == t, bv * 512 + pv, idxs)
        newrow = jnp.where(li == pv, -1.0, row)
        smap_ref[pl.ds(b, 1), :] = newrow
        rowmax = jnp.where(li == bv,
                           jnp.max(newrow, axis=1, keepdims=True), rowmax)
        return rowmax, vals, idxs

    vals0 = jnp.zeros((1, 512), jnp.float32)
    idxs0 = jnp.zeros((1, 512), jnp.int32)
    rowmax, vals, idxs = jax.lax.fori_loop(
        0, TOP_K, step, (rowmax, vals0, idxs0))
    vals_ref[...] = vals
    idxs_ref[...] = idxs


_SAMPLE_BLOCKS = 4
_SAMPLE_BK = 4096 // _SAMPLE_BLOCKS


def _sample_body(idx_ref, val_ref, feat_ref, kpts_ref, desc_ref, sc_ref,
                 acc_ref):
    j = pl.program_id(0)
    idx = idx_ref[...]
    vals = val_ref[...]
    kx = idx % W
    ky = idx // W
    x0 = kx // WS
    y0 = ky // WS
    wx = (kx % WS).astype(jnp.float32) / WS
    wy = (ky % WS).astype(jnp.float32) / WS
    x1 = jnp.minimum(x0 + 1, 63)
    y1 = jnp.minimum(y0 + 1, 63)
    c00 = y0 * 64 + x0
    c01 = y0 * 64 + x1
    c10 = y1 * 64 + x0
    c11 = y1 * 64 + x1
    w00 = (1.0 - wx) * (1.0 - wy)
    w01 = wx * (1.0 - wy)
    w10 = (1.0 - wx) * wy
    w11 = wx * wy
    ci = jax.lax.broadcasted_iota(jnp.int32, (512, _SAMPLE_BK), 1) \
        + j * _SAMPLE_BK
    Wm = jnp.where(ci == c00, w00, 0.0)
    Wm = Wm + jnp.where(ci == c01, w01, 0.0)
    Wm = Wm + jnp.where(ci == c10, w10, 0.0)
    Wm = Wm + jnp.where(ci == c11, w11, 0.0)
    part = jnp.dot(Wm, feat_ref[...], precision=HP,
                   preferred_element_type=jnp.float32)

    @pl.when(j == 0)
    def _():
        acc_ref[...] = part
        kpts_ref[...] = jnp.concatenate(
            [kx.astype(jnp.float32), ky.astype(jnp.float32)], axis=1)

    @pl.when(j > 0)
    def _():
        acc_ref[...] = acc_ref[...] + part

    @pl.when(j == _SAMPLE_BLOCKS - 1)
    def _():
        S = acc_ref[...]
        d = S[:, :64]
        rel = S[:, 64:65]
        dn = d / (jnp.sqrt(jnp.sum(d * d, axis=1, keepdims=True)) + 1e-8)
        sc = vals * rel
        sc = jnp.where(idx == 0, -1.0, sc)
        desc_ref[...] = dn
        sc_ref[...] = sc


def _f32(shape):
    return jax.ShapeDtypeStruct(shape, jnp.float32)


def kernel(grayscale_image, w1, b1, w2, b2, w3, b3, wf, bf,
           wk1, bk1, wk2, bk2, wk3, bk3, wh, bh):
    img = grayscale_image.reshape(H, W)

    xn = pl.pallas_call(_norm_body, out_shape=_f32((H, W)))(img)

    xp = jnp.pad(xn, ((0, 1), (0, 1)))
    taps1 = [xp[dy:dy + 511:2, dx:dx + 511:2]
             for dy in range(3) for dx in range(3)]
    p1 = jnp.stack(taps1, axis=-1).reshape(256 * 256, 9)
    W1 = w1.reshape(24, 9).T
    y1 = _mm_relu(p1, W1, b1.reshape(1, 24), 4096)

    x1 = y1.reshape(256, 256, 24)
    x1p = jnp.pad(x1, ((0, 1), (0, 1), (0, 0)))
    taps2 = [x1p[dy:dy + 255:2, dx:dx + 255:2, :]
             for dy in range(3) for dx in range(3)]
    p2 = jnp.stack(taps2, axis=2).reshape(128 * 128, 9 * 24)
    W2 = w2.transpose(2, 3, 1, 0).reshape(9 * 24, 24)
    y2 = _mm_relu(p2, W2, b2.reshape(1, 24), 4096)

    x2 = y2.reshape(128, 128, 24)
    x2p = jnp.pad(x2, ((0, 1), (0, 1), (0, 0)))
    taps3 = [x2p[dy:dy + 127:2, dx:dx + 127:2, :]
             for dy in range(3) for dx in range(3)]
    p3 = jnp.stack(taps3, axis=2).reshape(64 * 64, 9 * 24)
    W3 = w3.transpose(2, 3, 1, 0).reshape(9 * 24, 64)
    x3 = _mm_relu(p3, W3, b3.reshape(1, 64), 4096)

    u = xn.reshape(64, 8, 64, 8).transpose(1, 3, 0, 2).reshape(64, 4096).T

    Wf = wf.reshape(64, 64).T
    Wk1 = wk1.reshape(64, 64).T
    Wk2 = wk2.reshape(64, 64).T
    Wk3 = wk3.reshape(65, 64).T
    Wh = wh.reshape(1, 64).T
    sc64, feat = pl.pallas_call(
        _heads_body,
        out_shape=(_f32((4096, 64)), _f32((4096, 65))),
    )(x3, u, Wf, bf.reshape(1, 64), Wk1, bk1.reshape(1, 64),
      Wk2, bk2.reshape(1, 64), Wk3, bk3.reshape(1, 65), Wh, bh.reshape(1, 1))

    heat = sc64.reshape(64, 64, 8, 8).transpose(0, 2, 1, 3).reshape(H, W)
    hpad = jnp.pad(heat, 2, constant_values=-jnp.inf)

    vals, idxs = pl.pallas_call(
        _nms_topk_body,
        out_shape=(_f32((1, 512)), jax.ShapeDtypeStruct((1, 512), jnp.int32)),
        scratch_shapes=[pltpu.VMEM((512, 512), jnp.float32)],
    )(hpad)

    kpts, desc, scores = pl.pallas_call(
        _sample_body,
        grid=(_SAMPLE_BLOCKS,),
        in_specs=[pl.BlockSpec((512, 1), lambda j: (0, 0)),
                  pl.BlockSpec((512, 1), lambda j: (0, 0)),
                  pl.BlockSpec((_SAMPLE_BK, 65), lambda j: (j, 0))],
        out_specs=(pl.BlockSpec((512, 2), lambda j: (0, 0)),
                   pl.BlockSpec((512, 64), lambda j: (0, 0)),
                   pl.BlockSpec((512, 1), lambda j: (0, 0))),
        out_shape=(_f32((512, 2)), _f32((512, 64)), _f32((512, 1))),
        scratch_shapes=[pltpu.VMEM((512, 65), jnp.float32)],
    )(idxs.reshape(512, 1), vals.reshape(512, 1), feat)

    return kpts[:TOP_K], desc[:TOP_K], scores[:TOP_K, 0]
